# baseline (speedup 1.0000x reference)
import functools

import jax
import jax.numpy as jnp
from jax import lax
from jax.experimental import pallas as pl
from jax.experimental.pallas import tpu as pltpu
from jax.experimental.pallas import tpu_sc as plsc

_FDIM = 64
_EDIM = 1024

_NC = 2
_NS = 16
_NW = _NC * _NS
_LANES = 16
_NBUF = 8


def _sc_gather_t(idx_flat, table_t, n_rows):
    cols_per_w = n_rows // _NW
    n_groups = cols_per_w // _LANES
    mesh = plsc.VectorSubcoreMesh(core_axis_name="c", subcore_axis_name="s")

    @functools.partial(
        pl.kernel,
        mesh=mesh,
        out_type=jax.ShapeDtypeStruct((_FDIM, n_rows), jnp.float32),
        scratch_types=[
            pltpu.VMEM((cols_per_w + _LANES,), jnp.int32),
            pltpu.VMEM((_NBUF, _FDIM, 128), jnp.float32),
            pltpu.VMEM((_FDIM, cols_per_w), jnp.float32),
        ]
        + [pltpu.SemaphoreType.DMA] * _NBUF,
        compiler_params=pltpu.CompilerParams(
            needs_layout_passes=False, use_tc_tiling_on_sc=True
        ),
    )
    def gather_kernel(idx_hbm, table_hbm, out_hbm, idx_v, blk_v, stage_v, *sems):
        wid = lax.axis_index("s") * _NC + lax.axis_index("c")
        base = wid * cols_per_w
        pltpu.sync_copy(
            idx_hbm.at[pl.ds(base, cols_per_w)],
            idx_v.at[pl.ds(0, cols_per_w)],
        )
        lane = lax.iota(jnp.int32, _LANES)

        def group_ids(g):
            start = pl.multiple_of(g * _LANES, _LANES)
            chunk = idx_v[pl.ds(start, _LANES)]
            return [
                lax.reduce_sum(jnp.where(lane == k, chunk, 0), axes=(0,))
                for k in range(_LANES)
            ]

        def fetch(slot, tok_id):
            blk = pl.multiple_of((tok_id >> 7) * 128, 128)
            pltpu.make_async_copy(
                table_hbm.at[:, pl.ds(blk, 128)], blk_v.at[slot], sems[slot]
            ).start()

        def wait_slot(slot):
            pltpu.make_async_copy(
                table_hbm.at[:, pl.ds(0, 128)], blk_v.at[slot], sems[slot]
            ).wait()

        cur0 = group_ids(0)
        for t in range(_NBUF):
            fetch(t, cur0[t])

        def body(g, cur):
            nxt = group_ids(g + 1)
            for k in range(_LANES):
                slot = k % _NBUF
                wait_slot(slot)
                tvec = jnp.full((_LANES,), g * _LANES + k, jnp.int32)
                lvec = jnp.full((_LANES,), cur[k] & 127, jnp.int32)
                for q in range(_FDIM // _LANES):
                    jvec = lane + q * _LANES
                    vals = plsc.load_gather(blk_v.at[slot], [jvec, lvec])
                    plsc.store_scatter(stage_v, [jvec, tvec], vals)
                nid = cur[k + _NBUF] if k + _NBUF < _LANES else nxt[k + _NBUF - _LANES]

                @pl.when(g * _LANES + k + _NBUF < cols_per_w)
                def _():
                    fetch(slot, nid)

            return tuple(nxt)

        lax.fori_loop(0, n_groups, body, tuple(cur0), unroll=False)
        pltpu.sync_copy(stage_v, out_hbm.at[:, pl.ds(base, cols_per_w)])

    return gather_kernel(idx_flat, table_t)


def _tc_project_add(tok_t, factorized_table, pos_table, b, l):
    blk = 512
    l_blocks = l // blk

    def body(tok_ref, fac_ref, pos_ref, out_ref):
        out_ref[...] = (
            lax.dot_general(
                tok_ref[...],
                fac_ref[...],
                dimension_numbers=(((0,), (0,)), ((), ())),
                preferred_element_type=jnp.float32,
            )
            + pos_ref[...]
        )

    out = pl.pallas_call(
        body,
        grid=(l_blocks, b),
        in_specs=[
            pl.BlockSpec((_FDIM, blk), lambda i, j: (0, j * l_blocks + i)),
            pl.BlockSpec((_FDIM, _EDIM), lambda i, j: (0, 0)),
            pl.BlockSpec((blk, _EDIM), lambda i, j: (i, 0)),
        ],
        out_specs=pl.BlockSpec((blk, _EDIM), lambda i, j: (j * l_blocks + i, 0)),
        out_shape=jax.ShapeDtypeStruct((b * l, _EDIM), jnp.float32),
    )(tok_t, factorized_table, pos_table)
    return out


def kernel(inputs, token_table, factorized_table, segment_table, pos_table):
    b, l = inputs.shape
    n_rows = b * l
    half = n_rows // 2
    idx_flat = inputs.astype(jnp.int32).reshape(n_rows)
    table_t = token_table.T
    tok_a = _sc_gather_t(idx_flat[:half], table_t, half)
    tok_b = _sc_gather_t(idx_flat[half:], table_t, half)
    out_a = _tc_project_add(tok_a, factorized_table, pos_table, b // 2, l)
    out_b = _tc_project_add(tok_b, factorized_table, pos_table, b // 2, l)
    out = jnp.concatenate([out_a, out_b], axis=0)
    return out.reshape(b, l, _EDIM)


# speedup vs baseline: 1.2298x; 1.2298x over previous
import functools

import jax
import jax.numpy as jnp
from jax import lax
from jax.experimental import pallas as pl
from jax.experimental.pallas import tpu as pltpu
from jax.experimental.pallas import tpu_sc as plsc

_FDIM = 64
_EDIM = 1024

_NC = 2
_NS = 16
_NW = _NC * _NS
_LANES = 16
_NBUF = 8


def _sc_gather_t(idx_flat, table_t, n_rows):
    cols_per_w = n_rows // _NW
    n_groups = cols_per_w // _LANES
    mesh = plsc.VectorSubcoreMesh(core_axis_name="c", subcore_axis_name="s")

    @functools.partial(
        pl.kernel,
        mesh=mesh,
        out_type=jax.ShapeDtypeStruct((_FDIM, n_rows), jnp.float32),
        scratch_types=[
            pltpu.VMEM((cols_per_w + _LANES,), jnp.int32),
            pltpu.VMEM((_NBUF, _FDIM, 128), jnp.float32),
            pltpu.VMEM((_FDIM, cols_per_w), jnp.float32),
        ]
        + [pltpu.SemaphoreType.DMA] * _NBUF,
        compiler_params=pltpu.CompilerParams(
            needs_layout_passes=False, use_tc_tiling_on_sc=True
        ),
    )
    def gather_kernel(idx_hbm, table_hbm, out_hbm, idx_v, blk_v, stage_v, *sems):
        wid = lax.axis_index("s") * _NC + lax.axis_index("c")
        base = wid * cols_per_w
        pltpu.sync_copy(
            idx_hbm.at[pl.ds(base, cols_per_w)],
            idx_v.at[pl.ds(0, cols_per_w)],
        )
        lane = lax.iota(jnp.int32, _LANES)

        def group_ids(g):
            start = pl.multiple_of(g * _LANES, _LANES)
            chunk = idx_v[pl.ds(start, _LANES)]
            return [
                lax.reduce_sum(jnp.where(lane == k, chunk, 0), axes=(0,))
                for k in range(_LANES)
            ]

        def fetch(slot, tok_id):
            blk = pl.multiple_of((tok_id >> 7) * 128, 128)
            pltpu.make_async_copy(
                table_hbm.at[:, pl.ds(blk, 128)], blk_v.at[slot], sems[slot]
            ).start()

        def wait_slot(slot):
            pltpu.make_async_copy(
                table_hbm.at[:, pl.ds(0, 128)], blk_v.at[slot], sems[slot]
            ).wait()

        cur0 = group_ids(0)
        for t in range(_NBUF):
            fetch(t, cur0[t])

        def body(g, cur):
            nxt = group_ids(g + 1)
            for k in range(_LANES):
                slot = k % _NBUF
                wait_slot(slot)
                tvec = jnp.full((_LANES,), g * _LANES + k, jnp.int32)
                lvec = jnp.full((_LANES,), cur[k] & 127, jnp.int32)
                for q in range(_FDIM // _LANES):
                    jvec = lane + q * _LANES
                    vals = plsc.load_gather(blk_v.at[slot], [jvec, lvec])
                    plsc.store_scatter(stage_v, [jvec, tvec], vals)
                nid = cur[k + _NBUF] if k + _NBUF < _LANES else nxt[k + _NBUF - _LANES]

                @pl.when(g * _LANES + k + _NBUF < cols_per_w)
                def _():
                    fetch(slot, nid)

            return tuple(nxt)

        lax.fori_loop(0, n_groups, body, tuple(cur0), unroll=False)
        pltpu.sync_copy(stage_v, out_hbm.at[:, pl.ds(base, cols_per_w)])

    return gather_kernel(idx_flat, table_t)


def _tc_project_add(tok_t, factorized_table, pos_table, b, l):
    blk = 1024
    l_blocks = l // blk

    def body(tok_ref, fac_ref, pos_ref, out_ref):
        out_ref[...] = (
            lax.dot_general(
                tok_ref[...],
                fac_ref[...],
                dimension_numbers=(((0,), (0,)), ((), ())),
                preferred_element_type=jnp.float32,
            )
            + pos_ref[...]
        )

    out = pl.pallas_call(
        body,
        grid=(l_blocks, b),
        in_specs=[
            pl.BlockSpec((_FDIM, blk), lambda i, j: (0, j * l_blocks + i)),
            pl.BlockSpec((_FDIM, _EDIM), lambda i, j: (0, 0)),
            pl.BlockSpec((blk, _EDIM), lambda i, j: (i, 0)),
        ],
        out_specs=pl.BlockSpec((blk, _EDIM), lambda i, j: (j * l_blocks + i, 0)),
        out_shape=jax.ShapeDtypeStruct((b * l, _EDIM), jnp.float32),
    )(tok_t, factorized_table, pos_table)
    return out


def kernel(inputs, token_table, factorized_table, segment_table, pos_table):
    b, l = inputs.shape
    n_rows = b * l
    idx_flat = inputs.astype(jnp.int32).reshape(n_rows)
    tok_t = _sc_gather_t(idx_flat, token_table.T, n_rows)
    out = _tc_project_add(tok_t, factorized_table, pos_table, b, l)
    return out.reshape(b, l, _EDIM)
